# baseline (device time: 20169 ns/iter reference)
import functools

import jax
import jax.numpy as jnp
from jax import lax
from jax.experimental import pallas as pl
from jax.experimental.pallas import tpu as pltpu

N_DEV = 32
N_TOK = 256
D_IN = 128
D_OUT = 256
N_EXP = 64
EXP_PER_DEV = N_EXP // N_DEV
TOK_PER_DEV = N_TOK // N_DEV
CAP = 3
N_SLOT = EXP_PER_DEV * CAP


def kernel(x, router_W, route_idx, expert_W):
    del router_W

    def body(x_ref, idx_ref, w_ref, out_ref, send_buf, send_sems, recv_sem):
        my_pos = lax.axis_index("i")

        out_ref[...] = jnp.zeros((TOK_PER_DEV, D_OUT), jnp.float32)

        idx = idx_ref[:, :]
        e_iota = lax.broadcasted_iota(jnp.int32, (N_TOK, N_EXP), 1)
        oh = (idx == e_iota).astype(jnp.float32)
        row_i = lax.broadcasted_iota(jnp.int32, (N_TOK, N_TOK), 0)
        col_j = lax.broadcasted_iota(jnp.int32, (N_TOK, N_TOK), 1)
        lower = (col_j < row_i).astype(jnp.float32)
        csum = jnp.dot(lower, oh, preferred_element_type=jnp.float32)
        rank = jnp.sum(csum * oh, axis=1, keepdims=True)
        kept = (rank < float(CAP)).astype(jnp.float32)
        tok_iota = lax.broadcasted_iota(jnp.int32, (N_TOK, 1), 0)
        tok_f = tok_iota.astype(jnp.float32)

        barrier_sem = pltpu.get_barrier_semaphore()
        for k in range(N_DEV):
            pl.semaphore_signal(
                barrier_sem, inc=1,
                device_id=(k,), device_id_type=pl.DeviceIdType.MESH,
            )
        pl.semaphore_wait(barrier_sem, N_DEV)

        sends = []
        for le in range(EXP_PER_DEV):
            e = my_pos * EXP_PER_DEV + le
            is_e = (idx == e).astype(jnp.float32)
            served = kept * is_e
            for c in range(CAP):
                s = le * CAP + c
                sel = served * (rank == float(c)).astype(jnp.float32)
                exists = jnp.sum(sel) > 0.5
                t = jnp.sum(sel * tok_f).astype(jnp.int32)
                xg = lax.dot_general(
                    sel, x_ref[...], (((0,), (0,)), ((), ())),
                    preferred_element_type=jnp.float32,
                )
                y = jnp.dot(xg, w_ref[le], preferred_element_type=jnp.float32)
                send_buf[pl.ds(s, 1), :] = y
                dst_dev = t // TOK_PER_DEV
                dst_row = t % TOK_PER_DEV

                @pl.when(exists)
                def _(s=s, dst_dev=dst_dev, dst_row=dst_row):
                    rdma = pltpu.make_async_remote_copy(
                        src_ref=send_buf.at[pl.ds(s, 1)],
                        dst_ref=out_ref.at[pl.ds(dst_row, 1)],
                        send_sem=send_sems.at[s],
                        recv_sem=recv_sem,
                        device_id=(dst_dev,),
                        device_id_type=pl.DeviceIdType.MESH,
                    )
                    rdma.start()

                sends.append((s, exists))

        for r in range(TOK_PER_DEV):
            t_r = my_pos * TOK_PER_DEV + r
            onehot = (tok_iota == t_r).astype(jnp.float32)
            kept_r = jnp.sum(kept * onehot) > 0.5

            @pl.when(kept_r)
            def _(r=r):
                recv = pltpu.make_async_remote_copy(
                    src_ref=send_buf.at[pl.ds(0, 1)],
                    dst_ref=out_ref.at[pl.ds(r, 1)],
                    send_sem=send_sems.at[0],
                    recv_sem=recv_sem,
                    device_id=(0,),
                    device_id_type=pl.DeviceIdType.MESH,
                )
                recv.wait_recv()

        for s, exists in sends:
            @pl.when(exists)
            def _(s=s):
                rdma = pltpu.make_async_remote_copy(
                    src_ref=send_buf.at[pl.ds(s, 1)],
                    dst_ref=out_ref.at[pl.ds(0, 1)],
                    send_sem=send_sems.at[s],
                    recv_sem=recv_sem,
                    device_id=(0,),
                    device_id_type=pl.DeviceIdType.MESH,
                )
                rdma.wait_send()

        @functools.partial(pl.run_scoped, exit_sem=pltpu.SemaphoreType.REGULAR)
        def _(exit_sem):
            for k in range(N_DEV):
                pl.semaphore_signal(
                    exit_sem, inc=1,
                    device_id=(k,), device_id_type=pl.DeviceIdType.MESH,
                )
            pl.semaphore_wait(exit_sem, N_DEV)

    return pl.pallas_call(
        body,
        out_shape=jax.ShapeDtypeStruct((TOK_PER_DEV, D_OUT), jnp.float32),
        in_specs=[
            pl.BlockSpec(memory_space=pltpu.VMEM),
            pl.BlockSpec(memory_space=pltpu.VMEM),
            pl.BlockSpec(memory_space=pltpu.VMEM),
        ],
        out_specs=pl.BlockSpec(memory_space=pltpu.VMEM),
        scratch_shapes=[
            pltpu.VMEM((N_SLOT, D_OUT), jnp.float32),
            pltpu.SemaphoreType.DMA((N_SLOT,)),
            pltpu.SemaphoreType.DMA,
        ],
        compiler_params=pltpu.CompilerParams(collective_id=0),
    )(x, route_idx, expert_W)
